# Initial kernel scaffold; baseline (speedup 1.0000x reference)
import jax
import jax.numpy as jnp
from jax.experimental import pallas as pl


def kernel(unlabeled_inputs, W, b, unlabeled_targets):
    raise NotImplementedError("write your pallas kernel here")


# speedup vs baseline: 1.9244x; 1.9244x over previous
import jax
import jax.numpy as jnp
from jax.experimental import pallas as pl
from jax.experimental.pallas import tpu as pltpu

NUM_CLS = 100
TOPZ = 4096
N = 32768
D = 1024

BLK = 256
GRID1 = N // BLK
PRO_BLOCKS = (0, 43, 86)
R = N // 128
IDX_MASK = 0x7FFF


def _finish_scores(g, lgT, t_ref, key_ref, pl_ref):
    m = jnp.max(lgT, axis=0, keepdims=True)
    e = jnp.exp(lgT - m)
    s = jnp.sum(e, axis=0, keepdims=True)
    probs = e / s
    mp = jnp.max(probs, axis=0, keepdims=True)
    cidx = jax.lax.broadcasted_iota(jnp.int32, probs.shape, 0)
    am = jnp.min(jnp.where(probs == mp, cidx, 128), axis=0)
    am2 = am.reshape(1, BLK // 128, 128)
    match = (am2 == t_ref[...]).astype(jnp.int32)
    rows = jax.lax.broadcasted_iota(jnp.int32, (1, BLK // 128, 128), 1)
    cols = jax.lax.broadcasted_iota(jnp.int32, (1, BLK // 128, 128), 2)
    gidx = g * BLK + rows * 128 + cols
    key_ref[...] = mp.reshape(1, BLK // 128, 128)
    pl_ref[...] = gidx | (match << 15)


def _score_body(x_ref, w_ref, wt_ref, b_ref, bt_ref, t_ref, key_ref, pl_ref):
    g = pl.program_id(0)
    is_pro = (g == PRO_BLOCKS[0]) | (g == PRO_BLOCKS[1]) | (g == PRO_BLOCKS[2])

    @pl.when(jnp.logical_not(is_pro))
    def _():
        lg = jax.lax.dot_general(
            x_ref[...], w_ref[...], (((1,), (0,)), ((), ())),
            preferred_element_type=jnp.float32) + b_ref[0:1, :]
        _finish_scores(g, lg.T, t_ref, key_ref, pl_ref)

    @pl.when(is_pro)
    def _():
        lgT = jax.lax.dot_general(
            wt_ref[...], x_ref[...].T, (((1,), (0,)), ((), ())),
            preferred_element_type=jnp.float32) + bt_ref[:, 0:1]
        _finish_scores(g, lgT, t_ref, key_ref, pl_ref)


def _less(ka, pa, kb, pb):
    return (ka > kb) | ((ka == kb) & ((pa & IDX_MASK) < (pb & IDX_MASK)))


def _lane_stage(k, p, j, kk):
    lanes = jax.lax.broadcasted_iota(jnp.int32, k.shape, 1)
    hi = (lanes & j) != 0
    if kk < 7:
        desc = (lanes & (1 << kk)) != 0
    else:
        rows = jax.lax.broadcasted_iota(jnp.int32, k.shape, 0)
        desc = (rows & (1 << (kk - 7))) != 0
    pk = jnp.where(hi, pltpu.roll(k, j, 1), pltpu.roll(k, 128 - j, 1))
    pp = jnp.where(hi, pltpu.roll(p, j, 1), pltpu.roll(p, 128 - j, 1))
    stay = _less(k, p, pk, pp) ^ hi ^ desc
    return jnp.where(stay, k, pk), jnp.where(stay, p, pp)


def _row_stage(k, p, jr, kk):
    g_count = R // (2 * jr)
    k4 = k.reshape(g_count, 2, jr, 128)
    p4 = p.reshape(g_count, 2, jr, 128)
    a_k, b_k = k4[:, 0], k4[:, 1]
    a_p, b_p = p4[:, 0], p4[:, 1]
    kr = 1 << (kk - 7)
    gids = jax.lax.broadcasted_iota(jnp.int32, (g_count, 1, 1), 0)
    desc = ((gids * 2 * jr) & kr) != 0
    stay = _less(a_k, a_p, b_k, b_p) ^ desc
    out_a_k = jnp.where(stay, a_k, b_k)
    out_b_k = jnp.where(stay, b_k, a_k)
    out_a_p = jnp.where(stay, a_p, b_p)
    out_b_p = jnp.where(stay, b_p, a_p)
    k_out = jnp.stack([out_a_k, out_b_k], axis=1).reshape(R, 128)
    p_out = jnp.stack([out_a_p, out_b_p], axis=1).reshape(R, 128)
    return k_out, p_out


def _sort_body(key_ref, pl_ref, idx_ref, loss_ref, corr_ref, wod_ref):
    k = pltpu.bitcast(key_ref[...], jnp.int32)
    p = pl_ref[...]
    for kk in range(1, 16):
        for jb in range(kk - 1, -1, -1):
            j = 1 << jb
            if j < 128:
                k, p = _lane_stage(k, p, j, kk)
            else:
                k, p = _row_stage(k, p, j >> 7, kk)
    top_p = p[0 : TOPZ // 128, :]
    top_k = pltpu.bitcast(k[0 : TOPZ // 128, :], jnp.float32)
    idx_ref[...] = top_p & IDX_MASK
    loss_ref[...] = (jnp.sum(-jnp.log(top_k)) / TOPZ).reshape(1, 1)
    corr_ref[...] = jnp.sum(top_p >> 15).astype(jnp.float32).reshape(1, 1)
    wod_ref[...] = jnp.sum((top_p >= 0).astype(jnp.float32)).reshape(1, 1)


@jax.jit
def kernel(unlabeled_inputs, W, b, unlabeled_targets):
    w_pad = jnp.pad(W, ((0, 0), (0, 128 - NUM_CLS)))
    wt = jnp.pad(W.T, ((0, 128 - NUM_CLS), (0, 0)))
    b_pad = jnp.pad(b, (0, 128 - NUM_CLS), constant_values=-jnp.inf)
    b2d = jnp.broadcast_to(b_pad[None, :], (8, 128))
    bt2d = jnp.broadcast_to(b_pad[:, None], (128, 128))
    t3d = unlabeled_targets.astype(jnp.int32).reshape(GRID1, BLK // 128, 128)

    key2d, pay2d = pl.pallas_call(
        _score_body,
        grid=(GRID1,),
        in_specs=[
            pl.BlockSpec((BLK, D), lambda g: (g, 0)),
            pl.BlockSpec((D, 128), lambda g: (0, 0)),
            pl.BlockSpec((128, D), lambda g: (0, 0)),
            pl.BlockSpec((8, 128), lambda g: (0, 0)),
            pl.BlockSpec((128, 128), lambda g: (0, 0)),
            pl.BlockSpec((1, BLK // 128, 128), lambda g: (g, 0, 0)),
        ],
        out_specs=[
            pl.BlockSpec((1, BLK // 128, 128), lambda g: (g, 0, 0)),
            pl.BlockSpec((1, BLK // 128, 128), lambda g: (g, 0, 0)),
        ],
        out_shape=[
            jax.ShapeDtypeStruct((GRID1, BLK // 128, 128), jnp.float32),
            jax.ShapeDtypeStruct((GRID1, BLK // 128, 128), jnp.int32),
        ],
    )(unlabeled_inputs, w_pad, wt, b2d, bt2d, t3d)
    key2d = key2d.reshape(R, 128)
    pay2d = pay2d.reshape(R, 128)

    sel2d, loss, corr, wod = pl.pallas_call(
        _sort_body,
        out_shape=[
            jax.ShapeDtypeStruct((TOPZ // 128, 128), jnp.int32),
            jax.ShapeDtypeStruct((1, 1), jnp.float32),
            jax.ShapeDtypeStruct((1, 1), jnp.float32),
            jax.ShapeDtypeStruct((1, 1), jnp.float32),
        ],
    )(key2d, pay2d)

    return (loss[0, 0], corr[0, 0], wod[0, 0], sel2d.reshape(TOPZ))
